# Initial kernel scaffold; baseline (speedup 1.0000x reference)
import jax
import jax.numpy as jnp
from jax.experimental import pallas as pl


def kernel(positions, shifts, W_embed, W_radial, r0, atomic_numbers, edge_index):
    raise NotImplementedError("write your pallas kernel here")


# speedup vs baseline: 1.0612x; 1.0612x over previous
import functools
from math import factorial

import jax
import jax.numpy as jnp
import numpy as np
from jax.experimental import pallas as pl

ZS = (0, 1, 2, 3)
N_NODES = 10000
N_ATOM_BASIS = 2
CUTOFF = 5.5
N_RBF = 6
MAX_L = 2
N_MP = 1
P_CUT = 6


def _make_l_list(max_l):
    ls = []
    for l in range(max_l + 1):
        for lx in range(l, -1, -1):
            for ly in range(l - lx, -1, -1):
                ls.append((lx, ly, l - lx - ly))
    return ls


L_LIST = _make_l_list(MAX_L)
N_ANG = len(L_LIST)
L_OF_K = np.array([sum(t) for t in L_LIST])
SYM_COEF = np.array([factorial(sum(t)) / (factorial(t[0]) * factorial(t[1]) * factorial(t[2]))
                     for t in L_LIST], dtype=np.float32)


def _symmetrize_block(a_ref, a2_ref, out_ref):
    pieces = []
    cols = []
    for ref in (a_ref, a2_ref):
        A = ref[...]
        Ak = [A[:, 24 * k:24 * (k + 1)] for k in range(N_ANG)]
        nu1 = Ak[0]
        l0 = Ak[0] * Ak[0]
        l1 = Ak[1] * Ak[1] + Ak[2] * Ak[2] + Ak[3] * Ak[3]
        l2 = sum(float(SYM_COEF[k]) * Ak[k] * Ak[k] for k in range(4, N_ANG))
        cols.append((nu1, l0, l1, l2))
    for j in range(4):
        pieces.append(cols[0][j])
        pieces.append(cols[1][j])
    for j in range(8):
        out_ref[:, 24 * j:24 * (j + 1)] = pieces[j]


def _symmetrize(A, A2):
    N = A.shape[0]
    BLK = 1000
    out = pl.pallas_call(
        _symmetrize_block,
        out_shape=jax.ShapeDtypeStruct((N, 192), jnp.float32),
        grid=(N // BLK,),
        in_specs=[pl.BlockSpec((BLK, 240), lambda i: (i, 0)),
                  pl.BlockSpec((BLK, 240), lambda i: (i, 0))],
        out_specs=pl.BlockSpec((BLK, 192), lambda i: (i, 0)),
    )(A, A2)
    out = out.reshape(N, 8, 24).transpose(0, 2, 1)
    return out.reshape(N, N_RBF, 4, 4, 2)


def kernel(positions, shifts, W_embed, W_radial, r0, atomic_numbers, edge_index):
    onehot = (atomic_numbers[:, None] == jnp.asarray(ZS)[None, :]).astype(jnp.float32)
    emb = onehot @ W_embed
    sender = edge_index[0]
    receiver = edge_index[1]
    pos_s = positions[sender]
    pos_r = positions[receiver]
    emb_s = emb[sender]
    emb_r = emb[receiver]
    E = sender.shape[0]

    vec = pos_r - pos_s + shifts
    lengths = jnp.sqrt(jnp.sum(vec ** 2, axis=-1, keepdims=True) + 1e-12)
    unit = vec / lengths
    n = jnp.arange(1, N_RBF + 1, dtype=jnp.float32)
    rbf = jnp.sqrt(2.0 / CUTOFF) * jnp.sin(n * jnp.pi * lengths / CUTOFF) / lengths
    u = lengths / CUTOFF
    p = float(P_CUT)
    fcut = (1.0 - (p + 1.0) * (p + 2.0) / 2.0 * u ** P_CUT
            + p * (p + 2.0) * u ** (P_CUT + 1)
            - p * (p + 1.0) / 2.0 * u ** (P_CUT + 2)) * (u < 1.0)
    x, y, z = unit[:, 0], unit[:, 1], unit[:, 2]

    def pw(v, q):
        return jnp.ones_like(v) if q == 0 else v ** q
    ang = jnp.stack([pw(x, lx) * pw(y, ly) * pw(z, lz) for (lx, ly, lz) in L_LIST], axis=-1)
    Rl = jnp.einsum('er,lrs->els', rbf * fcut, W_radial)
    Rk = Rl[:, L_OF_K, :]
    P = Rk * ang[:, :, None]
    enc = (emb_s[:, :, None] * emb_r[:, None, :]).reshape(E, 4)
    edge_attri = P[:, None, :, :].transpose(0, 3, 2, 1) * enc[:, None, None, :]
    A = jax.ops.segment_sum(edge_attri, receiver, num_segments=N_NODES)
    decay = jnp.exp(-lengths / r0)
    msg = A[sender] * decay[:, :, None, None]
    A2 = A + jax.ops.segment_sum(msg, receiver, num_segments=N_NODES)
    Akm = A.transpose(0, 2, 1, 3).reshape(N_NODES, 240)
    A2km = A2.transpose(0, 2, 1, 3).reshape(N_NODES, 240)
    return _symmetrize(Akm, A2km)


# speedup vs baseline: 32.1926x; 30.3373x over previous
import functools
from math import factorial, pi, sqrt

import jax
import jax.numpy as jnp
import numpy as np
from jax import lax
from jax.experimental import pallas as pl
from jax.experimental.pallas import tpu as pltpu
from jax.experimental.pallas import tpu_sc as plsc

N_NODES = 10000
NP = 10240
E = 160000
CUTOFF = 5.5
N_RBF = 6
MAX_L = 2
N_ANG = 10
P_CUT = 6
NC, NS = 2, 16
EP = 163840
ROWS_PT = NP // NS


def _make_l_list(max_l):
    ls = []
    for l in range(max_l + 1):
        for lx in range(l, -1, -1):
            for ly in range(l - lx, -1, -1):
                ls.append((lx, ly, l - lx - ly))
    return ls


L_LIST = _make_l_list(MAX_L)
L_OF_K = [sum(t) for t in L_LIST]
SYM_COEF = [factorial(sum(t)) / (factorial(t[0]) * factorial(t[1]) * factorial(t[2]))
            for t in L_LIST]


def _tc0_body(pos_ref, an_ref, wemb_ref, out_ref):
    B = pos_ref.shape[0]
    out_ref[:, 0:3] = pos_ref[...]
    z = an_ref[...]
    for j in range(2):
        acc = jnp.zeros((B, 1), jnp.float32)
        for zv in range(4):
            acc = acc + jnp.where(z == zv, 1.0, 0.0) * wemb_ref[zv, j]
        out_ref[:, 3 + j:4 + j] = acc
    out_ref[:, 5:128] = jnp.zeros((B, 123), jnp.float32)


def _tc0(pos_pad, an_pad, W_embed):
    return pl.pallas_call(
        _tc0_body,
        out_shape=jax.ShapeDtypeStruct((NP, 128), jnp.float32),
        grid=(NP // 1280,),
        in_specs=[pl.BlockSpec((1280, 3), lambda i: (i, 0)),
                  pl.BlockSpec((1280, 1), lambda i: (i, 0)),
                  pl.BlockSpec((4, 2), lambda i: (0, 0))],
        out_specs=pl.BlockSpec((1280, 128), lambda i: (i, 0)),
    )(pos_pad, an_pad, W_embed)


def _sc0_body(table_hbm, snd_hbm, rcv_hbm, gs_hbm, gr_hbm,
              idx_v, rows_v, idx_t, rows_t, sem):
    wid = lax.axis_index("s") * NC + lax.axis_index("c")
    base0 = wid * (E // (NC * NS))

    for idx_hbm, out_hbm in ((snd_hbm, gs_hbm), (rcv_hbm, gr_hbm)):
        def blk(j, _):
            b = base0 + j * 128
            pltpu.sync_copy(idx_hbm.at[pl.ds(b, 128)], idx_v)
            pltpu.async_copy(table_hbm.at[idx_v], rows_v, sem).wait()
            pltpu.sync_copy(rows_v, out_hbm.at[pl.ds(b, 128)])
            return 0
        lax.fori_loop(0, 39, blk, 0)
        b = base0 + 39 * 128
        pltpu.sync_copy(idx_hbm.at[pl.ds(b, 8)], idx_t)
        pltpu.async_copy(table_hbm.at[idx_t], rows_t, sem).wait()
        pltpu.sync_copy(rows_t, out_hbm.at[pl.ds(b, 8)])


def _sc0(table, sender, receiver):
    mesh = plsc.VectorSubcoreMesh(core_axis_name="c", subcore_axis_name="s")
    f = pl.kernel(
        _sc0_body,
        out_type=[jax.ShapeDtypeStruct((E, 128), jnp.float32),
                  jax.ShapeDtypeStruct((E, 128), jnp.float32)],
        mesh=mesh,
        scratch_types=[pltpu.VMEM((128,), jnp.int32),
                       pltpu.VMEM((128, 128), jnp.float32),
                       pltpu.VMEM((8,), jnp.int32),
                       pltpu.VMEM((8, 128), jnp.float32),
                       pltpu.SemaphoreType.DMA],
    )
    return f(table, sender, receiver)


def _tc1_body(gsT_ref, grT_ref, shT_ref, wr_ref, r0_ref, payT_ref, decT_ref):
    px, py, pz = gsT_ref[:, 0], gsT_ref[:, 1], gsT_ref[:, 2]
    ea = (gsT_ref[:, 3], gsT_ref[:, 4])
    qx = grT_ref[:, 0] - px + shT_ref[:, 0]
    qy = grT_ref[:, 1] - py + shT_ref[:, 1]
    qz = grT_ref[:, 2] - pz + shT_ref[:, 2]
    d2 = qx * qx + qy * qy + qz * qz + 1e-12
    ln = jnp.sqrt(d2)
    inv = 1.0 / ln
    ux, uy, uz = qx * inv, qy * inv, qz * inv
    u = ln * (1.0 / CUTOFF)
    p = float(P_CUT)
    u2 = u * u
    u6 = u2 * u2 * u2
    fcut = (1.0 - (p + 1.0) * (p + 2.0) / 2.0 * u6
            + p * (p + 2.0) * u6 * u
            - p * (p + 1.0) / 2.0 * u6 * u2)
    fcut = jnp.where(u < 1.0, fcut, 0.0)
    args = jnp.concatenate([(float(r) * pi / CUTOFF) * ln for r in range(1, 7)], axis=0)
    sins = jnp.sin(args)
    scale = sqrt(2.0 / CUTOFF)
    nb = ln.shape[0]
    rbfw = [scale * sins[nb * r:nb * (r + 1)] * inv * fcut for r in range(6)]
    Rl = [[sum(rbfw[r] * wr_ref[l, r, s] for r in range(6)) for s in range(6)]
          for l in range(3)]
    one = jnp.ones_like(ux)

    def pw(v, q):
        if q == 0:
            return one
        out = v
        for _ in range(q - 1):
            out = out * v
        return out
    ang = [pw(ux, lx) * pw(uy, ly) * pw(uz, lz) for (lx, ly, lz) in L_LIST]
    zero = jnp.zeros_like(ux)
    for k in range(N_ANG):
        half, k2 = divmod(k, 5)
        for s in range(6):
            P = ang[k] * Rl[L_OF_K[k]][s]
            for a in range(2):
                row = 64 * half + (k2 * 6 + s) * 2 + a
                payT_ref[:, row, :] = P * ea[a]
    for half in range(2):
        for pad in range(60, 64):
            payT_ref[:, 64 * half + pad, :] = zero
    decT_ref[:, 0, :] = jnp.exp(-ln * (1.0 / r0_ref[0, 0]))


def _tc1(gsT, grT, shT, W_radial, r0_2d):
    R = E // 128
    return pl.pallas_call(
        _tc1_body,
        out_shape=[jax.ShapeDtypeStruct((R, 128, 128), jnp.float32),
                   jax.ShapeDtypeStruct((R, 1, 128), jnp.float32)],
        grid=(R // 10,),
        in_specs=[pl.BlockSpec((10, 5, 128), lambda i: (i, 0, 0)),
                  pl.BlockSpec((10, 5, 128), lambda i: (i, 0, 0)),
                  pl.BlockSpec((10, 3, 128), lambda i: (i, 0, 0)),
                  pl.BlockSpec((3, 6, 6), lambda i: (0, 0, 0)),
                  pl.BlockSpec((1, 1), lambda i: (0, 0))],
        out_specs=[pl.BlockSpec((10, 128, 128), lambda i: (i, 0, 0)),
                   pl.BlockSpec((10, 1, 128), lambda i: (i, 0, 0))],
    )(gsT, grT, shT, W_radial, r0_2d)


def _zero_stage(stage, rows, cols):
    z16 = jnp.zeros((16,), jnp.float32)

    def row(i, _):
        for q in range(cols // 16):
            stage[i, pl.ds(16 * q, 16)] = z16
        return 0
    lax.fori_loop(0, rows, row, 0)


def _sc1_body(pay_hbm, rcv_hbm, t1_hbm,
              idx2, buf, sem, t1_sh):
    c = lax.axis_index("c")
    s = lax.axis_index("s")
    rbase = s * ROWS_PT
    _zero_stage(buf, 128, 128)
    for t in range(ROWS_PT // 128):
        pltpu.sync_copy(buf, t1_sh.at[pl.ds(rbase + 128 * t, 128)])
    plsc.subcore_barrier()

    sep = EP // (NC * NS)
    ebase = c * (EP // NC) + s * sep

    def blk(j, _):
        off = ebase + j * 128
        pltpu.sync_copy(rcv_hbm.at[pl.ds(off, 128)], idx2.at[0])
        pltpu.sync_copy(pay_hbm.at[pl.ds(off, 128)], buf)
        pltpu.sync_copy(buf, t1_sh.at[idx2.at[0]], add=True)
        return 0
    lax.fori_loop(0, sep // 128, blk, 0)

    plsc.subcore_barrier()
    pltpu.sync_copy(t1_sh.at[pl.ds(rbase, ROWS_PT)],
                    t1_hbm.at[pl.ds(c * NP + rbase, ROWS_PT)])


def _sc1(pay_flat, receiver):
    mesh = plsc.VectorSubcoreMesh(core_axis_name="c", subcore_axis_name="s")
    f = pl.kernel(
        _sc1_body,
        out_type=jax.ShapeDtypeStruct((2 * NP, 128), jnp.float32),
        mesh=mesh,
        scratch_types=[pltpu.VMEM((1, 128), jnp.int32),
                       pltpu.VMEM((128, 128), jnp.float32),
                       pltpu.SemaphoreType.DMA,
                       pltpu.VMEM_SHARED((NP, 128), jnp.float32)],
    )
    return f(pay_flat, receiver)


def _tc3_body(t1_ref, tab_ref, g_ref):
    t = t1_ref[0] + t1_ref[1]
    e0 = tab_ref[:, 3:4]
    e1 = tab_ref[:, 4:5]
    th0 = t[:, 0:64]
    th1 = t[:, 64:128]
    g_ref[0, :, 0:64] = th0 * e0
    g_ref[0, :, 64:128] = th0 * e1
    g_ref[1, :, 0:64] = th1 * e0
    g_ref[1, :, 64:128] = th1 * e1


def _tc3(T1cat, table):
    return pl.pallas_call(
        _tc3_body,
        out_shape=jax.ShapeDtypeStruct((2, NP, 128), jnp.float32),
        grid=(NP // 1280,),
        in_specs=[pl.BlockSpec((2, 1280, 128), lambda i: (0, i, 0)),
                  pl.BlockSpec((1280, 128), lambda i: (i, 0))],
        out_specs=pl.BlockSpec((2, 1280, 128), lambda i: (0, i, 0)),
    )(T1cat, table)


def _sc2_body(g_hbm, snd_hbm, rcv_hbm, dec_hbm, a2_hbm,
              idxs, idxr2, dbuf, buf, sem, a2_sh):
    c = lax.axis_index("c")
    s = lax.axis_index("s")
    rbase = s * ROWS_PT
    _zero_stage(buf, 128, 128)
    for t in range(ROWS_PT // 128):
        pltpu.sync_copy(buf, a2_sh.at[pl.ds(rbase + 128 * t, 128)])
    plsc.subcore_barrier()

    sep = EP // NS
    ebase = s * sep
    coff = c * NP

    def blk(j, _):
        off = ebase + j * 128
        pltpu.sync_copy(snd_hbm.at[pl.ds(off, 128)], idxs)
        pltpu.sync_copy(rcv_hbm.at[pl.ds(off, 128)], idxr2.at[0])
        pltpu.sync_copy(dec_hbm.at[pl.ds(off, 128)], dbuf)
        for q in range(8):
            idxs[pl.ds(16 * q, 16)] = idxs[pl.ds(16 * q, 16)] + coff
        pltpu.async_copy(g_hbm.at[idxs], buf, sem).wait()

        def emul(e, _):
            dv = dbuf[e, pl.ds(0, 16)]
            for q in range(8):
                buf[e, pl.ds(16 * q, 16)] = buf[e, pl.ds(16 * q, 16)] * dv
            return 0
        lax.fori_loop(0, 128, emul, 0)
        pltpu.sync_copy(buf, a2_sh.at[idxr2.at[0]], add=True)
        return 0
    lax.fori_loop(0, sep // 128, blk, 0)

    plsc.subcore_barrier()
    pltpu.sync_copy(a2_sh.at[pl.ds(rbase, ROWS_PT)],
                    a2_hbm.at[pl.ds(coff + rbase, ROWS_PT)])


def _sc2(G_flat, sender, receiver, decay):
    mesh = plsc.VectorSubcoreMesh(core_axis_name="c", subcore_axis_name="s")
    f = pl.kernel(
        _sc2_body,
        out_type=jax.ShapeDtypeStruct((2 * NP, 128), jnp.float32),
        mesh=mesh,
        scratch_types=[pltpu.VMEM((128,), jnp.int32),
                       pltpu.VMEM((1, 128), jnp.int32),
                       pltpu.VMEM((128, 16), jnp.float32),
                       pltpu.VMEM((128, 128), jnp.float32),
                       pltpu.SemaphoreType.DMA,
                       pltpu.VMEM_SHARED((NP, 128), jnp.float32)],
    )
    return f(G_flat, sender, receiver, decay)


def _tc4_body(t1_ref, a2_ref, tab_ref, out_ref):
    emb = (tab_ref[:, 3:4], tab_ref[:, 4:5])
    t1s = t1_ref[0] + t1_ref[1]
    for b in range(2):
        A1 = []
        A2 = []
        for k in range(N_ANG):
            c01, k2 = divmod(k, 5)
            t12 = t1s[:, 64 * c01 + 12 * k2:64 * c01 + 12 * (k2 + 1)]
            a1 = t12 * emb[b]
            a2 = a1 + a2_ref[c01, :, 64 * b + 12 * k2:64 * b + 12 * (k2 + 1)]
            A1.append(a1)
            A2.append(a2)
        for mp, A in ((0, A1), (1, A2)):
            nu1 = A[0]
            l0 = A[0] * A[0]
            l1 = A[1] * A[1] + A[2] * A[2] + A[3] * A[3]
            l2 = sum(float(SYM_COEF[k]) * A[k] * A[k] for k in range(4, N_ANG))
            for nb, piece in enumerate((nu1, l0, l1, l2)):
                pidx = b * 8 + nb * 2 + mp
                out_ref[:, 12 * pidx:12 * (pidx + 1)] = piece


def _tc4(T1cat, A2cat, table):
    return pl.pallas_call(
        _tc4_body,
        out_shape=jax.ShapeDtypeStruct((NP, 192), jnp.float32),
        grid=(NP // 1280,),
        in_specs=[pl.BlockSpec((2, 1280, 128), lambda i: (0, i, 0)),
                  pl.BlockSpec((2, 1280, 128), lambda i: (0, i, 0)),
                  pl.BlockSpec((1280, 128), lambda i: (i, 0))],
        out_specs=pl.BlockSpec((1280, 192), lambda i: (i, 0)),
    )(T1cat, A2cat, table)


def kernel(positions, shifts, W_embed, W_radial, r0, atomic_numbers, edge_index):
    sender = edge_index[0].astype(jnp.int32)
    receiver = edge_index[1].astype(jnp.int32)
    pos_pad = jnp.pad(positions, ((0, NP - N_NODES), (0, 0)))
    an_pad = jnp.pad(atomic_numbers.astype(jnp.int32), (0, NP - N_NODES))[:, None]

    table = _tc0(pos_pad, an_pad, W_embed)
    gs, gr = _sc0(table, sender, receiver)

    R = E // 128
    gsT = gs.reshape(R, 128, 128).transpose(0, 2, 1)[:, :5, :]
    grT = gr.reshape(R, 128, 128).transpose(0, 2, 1)[:, :5, :]
    shT = shifts.reshape(R, 128, 3).transpose(0, 2, 1)
    r0_2d = r0.reshape(1, 1)
    payT, decT = _tc1(gsT, grT, shT, W_radial, r0_2d)

    pay_pad = jnp.pad(payT.transpose(0, 2, 1).reshape(E, 128),
                      ((0, EP - E), (0, 0)))
    decay = decT.reshape(E)
    snd_pad = jnp.pad(sender, (0, EP - E))
    rcv_pad = jnp.pad(receiver, (0, EP - E))
    dec_rep = jnp.pad(jnp.broadcast_to(decay[:, None], (E, 16)),
                      ((0, EP - E), (0, 0)))

    T1cat = _sc1(pay_pad, rcv_pad).reshape(2, NP, 128)
    G = _tc3(T1cat, table)
    A2cat = _sc2(G.reshape(2 * NP, 128), snd_pad, rcv_pad,
                 dec_rep).reshape(2, NP, 128)

    out_p = _tc4(T1cat, A2cat, table)
    out = out_p[:N_NODES].reshape(N_NODES, 2, 4, 2, 6, 2)
    out = out.transpose(0, 4, 5, 1, 2, 3)
    return out.reshape(N_NODES, N_RBF, 4, 4, 2)
